# Initial kernel scaffold; baseline (speedup 1.0000x reference)
import jax
import jax.numpy as jnp
from jax.experimental import pallas as pl


def kernel(x, edge_index, W1, b1, W2, b2, Wg1, bg1, Wg2, bg2, Wc, bc):
    raise NotImplementedError("write your pallas kernel here")


# speedup vs baseline: 13.3284x; 13.3284x over previous
import functools
import jax
import jax.numpy as jnp
from jax import lax
from jax.experimental import pallas as pl
from jax.experimental.pallas import tpu as pltpu
from jax.experimental.pallas import tpu_sc as plsc

N = 65536
NB = 2048
ROWS = 8192
FC = 16
NCHUNK = 4


def _front_body(xp_ref, w1_ref, b1_ref, w2_ref, b2_ref, out_ref):
    xp = xp_ref[0]
    c1 = []
    for p in range(4):
        a = jnp.zeros((16, NB), jnp.float32)
        for k in range(5):
            d = p + k - 2
            r = d % 4
            q = (d - r) // 4
            a = a + jnp.dot(w1_ref[k], xp[r, :, 1 + q:1 + q + NB],
                            preferred_element_type=jnp.float32)
        c1.append(jnp.maximum(a + b1_ref[...], 0.0))
    m = [jnp.maximum(c1[0], c1[1]), jnp.maximum(c1[2], c1[3])]
    zc = jnp.zeros((16, 1), jnp.float32)
    ph = [jnp.concatenate([zc, m[0], zc], axis=1),
          jnp.concatenate([zc, m[1], zc], axis=1)]
    c2 = []
    for p in range(2):
        a = jnp.zeros((32, NB), jnp.float32)
        for k in range(5):
            d = p + k - 2
            r = d % 2
            q = (d - r) // 2
            a = a + jnp.dot(w2_ref[k], ph[r][:, 1 + q:1 + q + NB],
                            preferred_element_type=jnp.float32)
        c2.append(jnp.maximum(a + b2_ref[...], 0.0))
    h2 = jnp.maximum(c2[0], c2[1])
    out_ref[...] = jnp.transpose(h2, (1, 0))


def _front_call(x4p, w1s, b1c, w2s, b2c):
    return pl.pallas_call(
        _front_body,
        grid=(32,),
        in_specs=[
            pl.BlockSpec((1, 4, 64, 2050), lambda b: (b, 0, 0, 0)),
            pl.BlockSpec((5, 16, 64), lambda b: (0, 0, 0)),
            pl.BlockSpec((16, 1), lambda b: (0, 0)),
            pl.BlockSpec((5, 32, 16), lambda b: (0, 0, 0)),
            pl.BlockSpec((32, 1), lambda b: (0, 0)),
        ],
        out_specs=pl.BlockSpec((NB, 32), lambda b: (b, 0)),
        out_shape=jax.ShapeDtypeStruct((N, 32), jnp.float32),
    )(x4p, w1s, b1c, w2s, b2c)


def _xw1_body(h0_ref, c0_ref, c1_ref, wg_ref, out_ref):
    dinv = lax.rsqrt(c0_ref[...] + c1_ref[...] + 1.0)
    xw = jnp.dot(h0_ref[...], wg_ref[...], preferred_element_type=jnp.float32)
    xs = dinv * xw
    for c in range(NCHUNK):
        out_ref[c] = xs[:, c * FC:(c + 1) * FC]


def _xw1_call(h0, c0, c1, wg1):
    return pl.pallas_call(
        _xw1_body,
        grid=(N // NB,),
        in_specs=[
            pl.BlockSpec((NB, 32), lambda i: (i, 0)),
            pl.BlockSpec((NB, 1), lambda i: (i, 0)),
            pl.BlockSpec((NB, 1), lambda i: (i, 0)),
            pl.BlockSpec((32, 64), lambda i: (0, 0)),
        ],
        out_specs=pl.BlockSpec((NCHUNK, NB, FC), lambda i: (0, i, 0)),
        out_shape=jax.ShapeDtypeStruct((NCHUNK, N, FC), jnp.float32),
    )(h0, c0, c1, wg1)


def _mid_body(acc_ref, xs_ref, c0_ref, c1_ref, bg_ref, wg_ref, out_ref):
    dinv = lax.rsqrt(c0_ref[...] + c1_ref[...] + 1.0)
    s = jnp.concatenate(
        [acc_ref[0, i] + acc_ref[1, i] + xs_ref[i] for i in range(NCHUNK)],
        axis=1)
    h = jnp.maximum(dinv * s + bg_ref[...], 0.0)
    y = jnp.dot(h, wg_ref[...], preferred_element_type=jnp.float32)
    ys = dinv * y
    for c in range(NCHUNK):
        out_ref[c] = ys[:, c * FC:(c + 1) * FC]


def _mid_call(acc1, xs1, c0, c1, bg1r, wg2):
    return pl.pallas_call(
        _mid_body,
        grid=(N // NB,),
        in_specs=[
            pl.BlockSpec((2, NCHUNK, NB, FC), lambda i: (0, 0, i, 0)),
            pl.BlockSpec((NCHUNK, NB, FC), lambda i: (0, i, 0)),
            pl.BlockSpec((NB, 1), lambda i: (i, 0)),
            pl.BlockSpec((NB, 1), lambda i: (i, 0)),
            pl.BlockSpec((1, 64), lambda i: (0, 0)),
            pl.BlockSpec((64, 64), lambda i: (0, 0)),
        ],
        out_specs=pl.BlockSpec((NCHUNK, NB, FC), lambda i: (0, i, 0)),
        out_shape=jax.ShapeDtypeStruct((NCHUNK, N, FC), jnp.float32),
    )(acc1, xs1, c0, c1, bg1r, wg2)


def _final_body(acc_ref, xs_ref, c0_ref, c1_ref, bg_ref, wc_ref, bc_ref,
                out_ref):
    b = pl.program_id(0)
    dinv = lax.rsqrt(c0_ref[...] + c1_ref[...] + 1.0)
    s = jnp.concatenate(
        [acc_ref[0, i] + acc_ref[1, i] + xs_ref[i] for i in range(NCHUNK)],
        axis=1)
    h = jnp.maximum(dinv * s + bg_ref[...], 0.0)
    hm = jnp.mean(h, axis=0, keepdims=True)
    out_ref[pl.ds(b, 1), :] = (
        jnp.dot(hm, wc_ref[...], preferred_element_type=jnp.float32)
        + bc_ref[...])


def _final_call(acc2, xs2, c0, c1, bg2r, wc, bcr):
    return pl.pallas_call(
        _final_body,
        grid=(32,),
        in_specs=[
            pl.BlockSpec((2, NCHUNK, NB, FC), lambda i: (0, 0, i, 0)),
            pl.BlockSpec((NCHUNK, NB, FC), lambda i: (0, i, 0)),
            pl.BlockSpec((NB, 1), lambda i: (i, 0)),
            pl.BlockSpec((NB, 1), lambda i: (i, 0)),
            pl.BlockSpec((1, 64), lambda i: (0, 0)),
            pl.BlockSpec((64, 16), lambda i: (0, 0)),
            pl.BlockSpec((1, 16), lambda i: (0, 0)),
        ],
        out_specs=pl.BlockSpec((32, 16), lambda i: (0, 0)),
        out_shape=jax.ShapeDtypeStruct((32, 16), jnp.float32),
    )(acc2, xs2, c0, c1, bg2r, wc, bcr)


def _deg_kernel(dst_hbm, counts_hbm, acc_sp, zbuf, ones_v, idx_v):
    c = lax.axis_index("c")
    s = lax.axis_index("s")

    def fill_z(i, _):
        zbuf[pl.ds(i * 16, 16)] = jnp.zeros((16,), jnp.float32)
        return 0
    lax.fori_loop(0, 256, fill_z, 0)
    for j in range(8):
        ones_v[pl.ds(j * 16, 16)] = jnp.full((16,), 1.0, jnp.float32)

    pltpu.sync_copy(zbuf, acc_sp.at[pl.ds(s * 4096, 4096)])
    plsc.subcore_barrier()

    def blk(i, _):
        base = c * 4096 + s * 256 + i * 16
        pltpu.sync_copy(dst_hbm.at[pl.ds(base, 16)], idx_v)

        def row(j, _):
            pltpu.sync_copy(ones_v, acc_sp.at[idx_v.at[j]], add=True)
            return 0
        return lax.fori_loop(0, 16, row, 0)
    lax.fori_loop(0, 16, blk, 0)
    plsc.subcore_barrier()
    pltpu.sync_copy(acc_sp.at[pl.ds(s * 4096, 4096)],
                    counts_hbm.at[c, pl.ds(s * 4096, 4096)])


def _deg_call(dst):
    f = functools.partial(
        pl.kernel,
        mesh=plsc.VectorSubcoreMesh(core_axis_name="c", subcore_axis_name="s"),
        compiler_params=pltpu.CompilerParams(use_tc_tiling_on_sc=False),
        out_type=jax.ShapeDtypeStruct((2, N), jnp.float32),
        scratch_types=[
            pltpu.VMEM_SHARED((N,), jnp.float32),
            pltpu.VMEM((4096,), jnp.float32),
            pltpu.VMEM((128,), jnp.float32),
            pltpu.VMEM((16, 128), jnp.int32),
        ],
    )(_deg_kernel)
    return f(dst)


def _scat_kernel(xs4_hbm, src_hbm, dst_hbm, out_hbm,
                 acc_sp, zbuf, sidx, didx, rbuf, sem0, sem1):
    c = lax.axis_index("c")
    s = lax.axis_index("s")
    sems = (sem0, sem1)

    def fill_z(i, _):
        zbuf[i] = jnp.zeros((16,), jnp.float32)
        return 0
    lax.fori_loop(0, 512, fill_z, 0)

    for f in range(NCHUNK):
        def zero(i, _):
            pltpu.sync_copy(zbuf, acc_sp.at[pl.ds(s * 4096 + i * 512, 512)])
            return 0
        lax.fori_loop(0, 8, zero, 0)
        plsc.subcore_barrier()

        def blk(i, _):
            base = c * 4096 + s * 256 + i * 16
            pltpu.sync_copy(src_hbm.at[pl.ds(base, 16)], sidx)
            pltpu.sync_copy(dst_hbm.at[pl.ds(base, 16)], didx)
            pending = pltpu.async_copy(
                xs4_hbm.at[f].at[sidx.at[0]], rbuf.at[0], sems[0])
            for j in range(16):
                pending.wait()
                if j < 15:
                    pending = pltpu.async_copy(
                        xs4_hbm.at[f].at[sidx.at[j + 1]],
                        rbuf.at[(j + 1) % 2], sems[(j + 1) % 2])
                pltpu.sync_copy(rbuf.at[j % 2], acc_sp.at[didx.at[j]],
                                add=True)
            return 0
        lax.fori_loop(0, 16, blk, 0)
        plsc.subcore_barrier()
        pltpu.sync_copy(acc_sp.at[pl.ds(s * 4096, 4096)],
                        out_hbm.at[c, f, pl.ds(s * 4096, 4096)])
        plsc.subcore_barrier()


def _scat_call(xs4, src, dst):
    f = functools.partial(
        pl.kernel,
        mesh=plsc.VectorSubcoreMesh(core_axis_name="c", subcore_axis_name="s"),
        compiler_params=pltpu.CompilerParams(use_tc_tiling_on_sc=False),
        out_type=jax.ShapeDtypeStruct((2, NCHUNK, N, FC), jnp.float32),
        scratch_types=[
            pltpu.VMEM_SHARED((N, FC), jnp.float32),
            pltpu.VMEM((512, FC), jnp.float32),
            pltpu.VMEM((16, 128), jnp.int32),
            pltpu.VMEM((16, 128), jnp.int32),
            pltpu.VMEM((2, 128, FC), jnp.float32),
            pltpu.SemaphoreType.DMA,
            pltpu.SemaphoreType.DMA,
        ],
    )(_scat_kernel)
    return f(xs4, src, dst)


def kernel(x, edge_index, W1, b1, W2, b2, Wg1, bg1, Wg2, bg2, Wc, bc):
    B, C, T = x.shape

    x4 = jnp.transpose(x.reshape(B, C, T // 4, 4), (0, 3, 1, 2))
    x4p = jnp.pad(x4, ((0, 0), (0, 0), (0, 0), (1, 1)))
    w1s = jnp.transpose(W1, (2, 0, 1))
    w2s = jnp.transpose(W2, (2, 0, 1))
    b1c = b1.reshape(16, 1)
    b2c = b2.reshape(32, 1)
    bg1r = bg1.reshape(1, 64)
    bg2r = bg2.reshape(1, 64)
    bcr = bc.reshape(1, 16)
    src = edge_index[0].reshape(ROWS, 128)
    dst = edge_index[1].reshape(ROWS, 128)

    counts = _deg_call(dst)
    c0 = counts[0].reshape(N, 1)
    c1 = counts[1].reshape(N, 1)

    h0 = _front_call(x4p, w1s, b1c, w2s, b2c)
    xs1 = _xw1_call(h0, c0, c1, Wg1)
    acc1 = _scat_call(xs1, src, dst)
    xs2 = _mid_call(acc1, xs1, c0, c1, bg1r, Wg2)
    acc2 = _scat_call(xs2, src, dst)
    return _final_call(acc2, xs2, c0, c1, bg2r, Wc, bcr)


# speedup vs baseline: 22.5333x; 1.6906x over previous
import functools
import jax
import jax.numpy as jnp
from jax import lax
from jax.experimental import pallas as pl
from jax.experimental.pallas import tpu as pltpu
from jax.experimental.pallas import tpu_sc as plsc

N = 65536
NB = 2048
ROWS = 8192
FC = 16
NCHUNK = 4


def _front_body(xp_ref, w1_ref, b1_ref, w2_ref, b2_ref, out_ref):
    xp = xp_ref[0]
    c1 = []
    for p in range(4):
        a = jnp.zeros((16, NB), jnp.float32)
        for k in range(5):
            d = p + k - 2
            r = d % 4
            q = (d - r) // 4
            a = a + jnp.dot(w1_ref[k], xp[r, :, 1 + q:1 + q + NB],
                            preferred_element_type=jnp.float32)
        c1.append(jnp.maximum(a + b1_ref[...], 0.0))
    m = [jnp.maximum(c1[0], c1[1]), jnp.maximum(c1[2], c1[3])]
    zc = jnp.zeros((16, 1), jnp.float32)
    ph = [jnp.concatenate([zc, m[0], zc], axis=1),
          jnp.concatenate([zc, m[1], zc], axis=1)]
    c2 = []
    for p in range(2):
        a = jnp.zeros((32, NB), jnp.float32)
        for k in range(5):
            d = p + k - 2
            r = d % 2
            q = (d - r) // 2
            a = a + jnp.dot(w2_ref[k], ph[r][:, 1 + q:1 + q + NB],
                            preferred_element_type=jnp.float32)
        c2.append(jnp.maximum(a + b2_ref[...], 0.0))
    h2 = jnp.maximum(c2[0], c2[1])
    out_ref[...] = jnp.transpose(h2, (1, 0))


def _front_call(x4p, w1s, b1c, w2s, b2c):
    return pl.pallas_call(
        _front_body,
        grid=(32,),
        in_specs=[
            pl.BlockSpec((1, 4, 64, 2050), lambda b: (b, 0, 0, 0)),
            pl.BlockSpec((5, 16, 64), lambda b: (0, 0, 0)),
            pl.BlockSpec((16, 1), lambda b: (0, 0)),
            pl.BlockSpec((5, 32, 16), lambda b: (0, 0, 0)),
            pl.BlockSpec((32, 1), lambda b: (0, 0)),
        ],
        out_specs=pl.BlockSpec((NB, 32), lambda b: (b, 0)),
        out_shape=jax.ShapeDtypeStruct((N, 32), jnp.float32),
    )(x4p, w1s, b1c, w2s, b2c)


def _xw1_body(h0_ref, c0_ref, c1_ref, wg_ref, out_ref):
    dinv = lax.rsqrt(c0_ref[...] + c1_ref[...] + 1.0)
    xw = jnp.dot(h0_ref[...], wg_ref[...], preferred_element_type=jnp.float32)
    xs = dinv * xw
    for c in range(NCHUNK):
        out_ref[c] = xs[:, c * FC:(c + 1) * FC]


def _xw1_call(h0, c0, c1, wg1):
    return pl.pallas_call(
        _xw1_body,
        grid=(N // NB,),
        in_specs=[
            pl.BlockSpec((NB, 32), lambda i: (i, 0)),
            pl.BlockSpec((NB, 1), lambda i: (i, 0)),
            pl.BlockSpec((NB, 1), lambda i: (i, 0)),
            pl.BlockSpec((32, 64), lambda i: (0, 0)),
        ],
        out_specs=pl.BlockSpec((NCHUNK, NB, FC), lambda i: (0, i, 0)),
        out_shape=jax.ShapeDtypeStruct((NCHUNK, N, FC), jnp.float32),
    )(h0, c0, c1, wg1)


def _mid_body(acc_ref, xs_ref, c0_ref, c1_ref, bg_ref, wg_ref, out_ref):
    dinv = lax.rsqrt(c0_ref[...] + c1_ref[...] + 1.0)
    s = jnp.concatenate(
        [acc_ref[0, i] + acc_ref[1, i] + xs_ref[i] for i in range(NCHUNK)],
        axis=1)
    h = jnp.maximum(dinv * s + bg_ref[...], 0.0)
    y = jnp.dot(h, wg_ref[...], preferred_element_type=jnp.float32)
    ys = dinv * y
    for c in range(NCHUNK):
        out_ref[c] = ys[:, c * FC:(c + 1) * FC]


def _mid_call(acc1, xs1, c0, c1, bg1r, wg2):
    return pl.pallas_call(
        _mid_body,
        grid=(N // NB,),
        in_specs=[
            pl.BlockSpec((2, NCHUNK, NB, FC), lambda i: (0, 0, i, 0)),
            pl.BlockSpec((NCHUNK, NB, FC), lambda i: (0, i, 0)),
            pl.BlockSpec((NB, 1), lambda i: (i, 0)),
            pl.BlockSpec((NB, 1), lambda i: (i, 0)),
            pl.BlockSpec((1, 64), lambda i: (0, 0)),
            pl.BlockSpec((64, 64), lambda i: (0, 0)),
        ],
        out_specs=pl.BlockSpec((NCHUNK, NB, FC), lambda i: (0, i, 0)),
        out_shape=jax.ShapeDtypeStruct((NCHUNK, N, FC), jnp.float32),
    )(acc1, xs1, c0, c1, bg1r, wg2)


def _final_body(acc_ref, xs_ref, c0_ref, c1_ref, bg_ref, wc_ref, bc_ref,
                out_ref):
    b = pl.program_id(0)
    dinv = lax.rsqrt(c0_ref[...] + c1_ref[...] + 1.0)
    s = jnp.concatenate(
        [acc_ref[0, i] + acc_ref[1, i] + xs_ref[i] for i in range(NCHUNK)],
        axis=1)
    h = jnp.maximum(dinv * s + bg_ref[...], 0.0)
    hm = jnp.mean(h, axis=0, keepdims=True)
    out_ref[pl.ds(b, 1), :] = (
        jnp.dot(hm, wc_ref[...], preferred_element_type=jnp.float32)
        + bc_ref[...])


def _final_call(acc2, xs2, c0, c1, bg2r, wc, bcr):
    return pl.pallas_call(
        _final_body,
        grid=(32,),
        in_specs=[
            pl.BlockSpec((2, NCHUNK, NB, FC), lambda i: (0, 0, i, 0)),
            pl.BlockSpec((NCHUNK, NB, FC), lambda i: (0, i, 0)),
            pl.BlockSpec((NB, 1), lambda i: (i, 0)),
            pl.BlockSpec((NB, 1), lambda i: (i, 0)),
            pl.BlockSpec((1, 64), lambda i: (0, 0)),
            pl.BlockSpec((64, 16), lambda i: (0, 0)),
            pl.BlockSpec((1, 16), lambda i: (0, 0)),
        ],
        out_specs=pl.BlockSpec((32, 16), lambda i: (0, 0)),
        out_shape=jax.ShapeDtypeStruct((32, 16), jnp.float32),
    )(acc2, xs2, c0, c1, bg2r, wc, bcr)


def _deg_kernel(dst_hbm, counts_hbm, acc_sp, zbuf, ones_v, idx_v):
    c = lax.axis_index("c")
    s = lax.axis_index("s")

    def fill_z(i, _):
        zbuf[pl.ds(i * 16, 16)] = jnp.zeros((16,), jnp.float32)
        return 0
    lax.fori_loop(0, 256, fill_z, 0)
    for j in range(8):
        ones_v[pl.ds(j * 16, 16)] = jnp.full((16,), 1.0, jnp.float32)

    pltpu.sync_copy(zbuf, acc_sp.at[pl.ds(s * 4096, 4096)])
    plsc.subcore_barrier()

    def blk(i, _):
        base = c * 4096 + s * 256 + i * 16
        pltpu.sync_copy(dst_hbm.at[pl.ds(base, 16)], idx_v)

        def row(j, _):
            pltpu.sync_copy(ones_v, acc_sp.at[idx_v.at[j]], add=True)
            return 0
        return lax.fori_loop(0, 16, row, 0)
    lax.fori_loop(0, 16, blk, 0)
    plsc.subcore_barrier()
    pltpu.sync_copy(acc_sp.at[pl.ds(s * 4096, 4096)],
                    counts_hbm.at[c, pl.ds(s * 4096, 4096)])


def _deg_call(dst):
    f = functools.partial(
        pl.kernel,
        mesh=plsc.VectorSubcoreMesh(core_axis_name="c", subcore_axis_name="s"),
        compiler_params=pltpu.CompilerParams(use_tc_tiling_on_sc=False),
        out_type=jax.ShapeDtypeStruct((2, N), jnp.float32),
        scratch_types=[
            pltpu.VMEM_SHARED((N,), jnp.float32),
            pltpu.VMEM((4096,), jnp.float32),
            pltpu.VMEM((128,), jnp.float32),
            pltpu.VMEM((16, 128), jnp.int32),
        ],
    )(_deg_kernel)
    return f(dst)


_NBUF = 6
_LOOK = 4


def _scat_kernel(xs4_hbm, src_hbm, dst_hbm, out_hbm,
                 acc_sp, zbuf, sidx, didx, rbuf, gsem, ssem, isem):
    c = lax.axis_index("c")
    s = lax.axis_index("s")
    tbase = c * 4096 + s * 256

    def fill_z(i, _):
        zbuf[i] = jnp.zeros((16,), jnp.float32)
        return 0
    lax.fori_loop(0, 1024, fill_z, 0)

    def issue_idx(i):
        pltpu.async_copy(src_hbm.at[pl.ds(tbase + i * 16, 16)],
                         sidx.at[i % 2], isem)
        pltpu.async_copy(dst_hbm.at[pl.ds(tbase + i * 16, 16)],
                         didx.at[i % 2], isem)

    def wait_idx(i):
        pltpu.make_async_copy(src_hbm.at[pl.ds(tbase + i * 16, 16)],
                              sidx.at[i % 2], isem).wait()
        pltpu.make_async_copy(dst_hbm.at[pl.ds(tbase + i * 16, 16)],
                              didx.at[i % 2], isem).wait()

    for f in range(NCHUNK):
        def zero(i, _):
            pltpu.sync_copy(zbuf, acc_sp.at[pl.ds(s * 4096 + i * 1024, 1024)])
            return 0
        lax.fori_loop(0, 4, zero, 0)
        issue_idx(0)
        plsc.subcore_barrier()

        def blk(i, _):
            ib = i % 2
            wait_idx(i)

            @pl.when(i < 15)
            def _():
                issue_idx(i + 1)

            def gather(t):
                k = t % _NBUF
                return pltpu.async_copy(
                    xs4_hbm.at[f].at[sidx.at[ib, t]], rbuf.at[k], gsem.at[k])

            gh = [None] * 16
            sh = [None] * 16
            for t in range(_LOOK):
                gh[t] = gather(t)
            for j in range(16):
                gh[j].wait()
                sh[j] = pltpu.async_copy(
                    rbuf.at[j % _NBUF], acc_sp.at[didx.at[ib, j]],
                    ssem.at[j % _NBUF], add=True)
                t = j + _LOOK
                if t < 16:
                    if t - _NBUF >= 0:
                        sh[t - _NBUF].wait()
                    gh[t] = gather(t)
            for j in range(16 - _NBUF, 16):
                sh[j].wait()
            return 0
        lax.fori_loop(0, 16, blk, 0)
        plsc.subcore_barrier()
        pltpu.sync_copy(acc_sp.at[pl.ds(s * 4096, 4096)],
                        out_hbm.at[c, f, pl.ds(s * 4096, 4096)])
        plsc.subcore_barrier()


def _scat_call(xs4, src, dst):
    f = functools.partial(
        pl.kernel,
        mesh=plsc.VectorSubcoreMesh(core_axis_name="c", subcore_axis_name="s"),
        compiler_params=pltpu.CompilerParams(use_tc_tiling_on_sc=False),
        out_type=jax.ShapeDtypeStruct((2, NCHUNK, N, FC), jnp.float32),
        scratch_types=[
            pltpu.VMEM_SHARED((N, FC), jnp.float32),
            pltpu.VMEM((1024, FC), jnp.float32),
            pltpu.VMEM((2, 16, 128), jnp.int32),
            pltpu.VMEM((2, 16, 128), jnp.int32),
            pltpu.VMEM((_NBUF, 128, FC), jnp.float32),
            pltpu.SemaphoreType.DMA((_NBUF,)),
            pltpu.SemaphoreType.DMA((_NBUF,)),
            pltpu.SemaphoreType.DMA,
        ],
    )(_scat_kernel)
    return f(xs4, src, dst)


def kernel(x, edge_index, W1, b1, W2, b2, Wg1, bg1, Wg2, bg2, Wc, bc):
    B, C, T = x.shape

    x4 = jnp.transpose(x.reshape(B, C, T // 4, 4), (0, 3, 1, 2))
    x4p = jnp.pad(x4, ((0, 0), (0, 0), (0, 0), (1, 1)))
    w1s = jnp.transpose(W1, (2, 0, 1))
    w2s = jnp.transpose(W2, (2, 0, 1))
    b1c = b1.reshape(16, 1)
    b2c = b2.reshape(32, 1)
    bg1r = bg1.reshape(1, 64)
    bg2r = bg2.reshape(1, 64)
    bcr = bc.reshape(1, 16)
    src = edge_index[0].reshape(ROWS, 128)
    dst = edge_index[1].reshape(ROWS, 128)

    counts = _deg_call(dst)
    c0 = counts[0].reshape(N, 1)
    c1 = counts[1].reshape(N, 1)

    h0 = _front_call(x4p, w1s, b1c, w2s, b2c)
    xs1 = _xw1_call(h0, c0, c1, Wg1)
    acc1 = _scat_call(xs1, src, dst)
    xs2 = _mid_call(acc1, xs1, c0, c1, bg1r, Wg2)
    acc2 = _scat_call(xs2, src, dst)
    return _final_call(acc2, xs2, c0, c1, bg2r, Wc, bcr)


# speedup vs baseline: 23.3373x; 1.0357x over previous
import functools
import jax
import jax.numpy as jnp
from jax import lax
from jax.experimental import pallas as pl
from jax.experimental.pallas import tpu as pltpu
from jax.experimental.pallas import tpu_sc as plsc

N = 65536
NB = 2048
ROWS = 8192
FC = 16
NCHUNK = 4


def _shift1(x, q):
    if q == 0:
        return x
    zc = jnp.zeros((x.shape[0], 1), jnp.float32)
    if q == -1:
        return jnp.concatenate([zc, x[:, :-1]], axis=1)
    return jnp.concatenate([x[:, 1:], zc], axis=1)


def _front_body(xp_ref, w1_ref, b1_ref, w2_ref, b2_ref, c0_ref, c1_ref,
                wg_ref, out_ref):
    xp = xp_ref[0]
    sh1 = {}
    for r in range(4):
        for q in (-1, 0, 1):
            sh1[(r, q)] = None
    c1 = []
    for p in range(4):
        a = jnp.zeros((16, NB), jnp.float32)
        for k in range(5):
            d = p + k - 2
            r = d % 4
            q = (d - r) // 4
            if sh1[(r, q)] is None:
                sh1[(r, q)] = _shift1(xp[r], q)
            a = a + jnp.dot(w1_ref[k], sh1[(r, q)],
                            preferred_element_type=jnp.float32)
        c1.append(jnp.maximum(a + b1_ref[...], 0.0))
    m = [jnp.maximum(c1[0], c1[1]), jnp.maximum(c1[2], c1[3])]
    sh2 = {}
    c2 = []
    for p in range(2):
        a = jnp.zeros((32, NB), jnp.float32)
        for k in range(5):
            d = p + k - 2
            r = d % 2
            q = (d - r) // 2
            if (r, q) not in sh2:
                sh2[(r, q)] = _shift1(m[r], q)
            a = a + jnp.dot(w2_ref[k], sh2[(r, q)],
                            preferred_element_type=jnp.float32)
        c2.append(jnp.maximum(a + b2_ref[...], 0.0))
    h2 = jnp.maximum(c2[0], c2[1])
    h0b = jnp.transpose(h2, (1, 0))
    dinv = lax.rsqrt(c0_ref[...] + c1_ref[...] + 1.0)
    xs = dinv * jnp.dot(h0b, wg_ref[...], preferred_element_type=jnp.float32)
    for c in range(NCHUNK):
        out_ref[c] = xs[:, c * FC:(c + 1) * FC]


def _front_call(x4, w1s, b1c, w2s, b2c, c0, c1, wg1):
    return pl.pallas_call(
        _front_body,
        grid=(32,),
        in_specs=[
            pl.BlockSpec((1, 4, 64, NB), lambda b: (b, 0, 0, 0)),
            pl.BlockSpec((5, 16, 64), lambda b: (0, 0, 0)),
            pl.BlockSpec((16, 1), lambda b: (0, 0)),
            pl.BlockSpec((5, 32, 16), lambda b: (0, 0, 0)),
            pl.BlockSpec((32, 1), lambda b: (0, 0)),
            pl.BlockSpec((NB, 1), lambda b: (b, 0)),
            pl.BlockSpec((NB, 1), lambda b: (b, 0)),
            pl.BlockSpec((32, 64), lambda b: (0, 0)),
        ],
        out_specs=pl.BlockSpec((NCHUNK, NB, FC), lambda b: (0, b, 0)),
        out_shape=jax.ShapeDtypeStruct((NCHUNK, N, FC), jnp.float32),
    )(x4, w1s, b1c, w2s, b2c, c0, c1, wg1)


def _mid_body(acc_ref, xs_ref, c0_ref, c1_ref, bg_ref, wg_ref, out_ref):
    dinv = lax.rsqrt(c0_ref[...] + c1_ref[...] + 1.0)
    s = jnp.concatenate(
        [acc_ref[0, i] + acc_ref[1, i] + xs_ref[i] for i in range(NCHUNK)],
        axis=1)
    h = jnp.maximum(dinv * s + bg_ref[...], 0.0)
    y = jnp.dot(h, wg_ref[...], preferred_element_type=jnp.float32)
    ys = dinv * y
    for c in range(NCHUNK):
        out_ref[c] = ys[:, c * FC:(c + 1) * FC]


def _mid_call(acc1, xs1, c0, c1, bg1r, wg2):
    return pl.pallas_call(
        _mid_body,
        grid=(N // NB,),
        in_specs=[
            pl.BlockSpec((2, NCHUNK, NB, FC), lambda i: (0, 0, i, 0)),
            pl.BlockSpec((NCHUNK, NB, FC), lambda i: (0, i, 0)),
            pl.BlockSpec((NB, 1), lambda i: (i, 0)),
            pl.BlockSpec((NB, 1), lambda i: (i, 0)),
            pl.BlockSpec((1, 64), lambda i: (0, 0)),
            pl.BlockSpec((64, 64), lambda i: (0, 0)),
        ],
        out_specs=pl.BlockSpec((NCHUNK, NB, FC), lambda i: (0, i, 0)),
        out_shape=jax.ShapeDtypeStruct((NCHUNK, N, FC), jnp.float32),
    )(acc1, xs1, c0, c1, bg1r, wg2)


def _final_body(acc_ref, xs_ref, c0_ref, c1_ref, bg_ref, wc_ref, bc_ref,
                out_ref):
    b = pl.program_id(0)
    dinv = lax.rsqrt(c0_ref[...] + c1_ref[...] + 1.0)
    s = jnp.concatenate(
        [acc_ref[0, i] + acc_ref[1, i] + xs_ref[i] for i in range(NCHUNK)],
        axis=1)
    h = jnp.maximum(dinv * s + bg_ref[...], 0.0)
    hm = jnp.mean(h, axis=0, keepdims=True)
    out_ref[pl.ds(b, 1), :] = (
        jnp.dot(hm, wc_ref[...], preferred_element_type=jnp.float32)
        + bc_ref[...])


def _final_call(acc2, xs2, c0, c1, bg2r, wc, bcr):
    return pl.pallas_call(
        _final_body,
        grid=(32,),
        in_specs=[
            pl.BlockSpec((2, NCHUNK, NB, FC), lambda i: (0, 0, i, 0)),
            pl.BlockSpec((NCHUNK, NB, FC), lambda i: (0, i, 0)),
            pl.BlockSpec((NB, 1), lambda i: (i, 0)),
            pl.BlockSpec((NB, 1), lambda i: (i, 0)),
            pl.BlockSpec((1, 64), lambda i: (0, 0)),
            pl.BlockSpec((64, 16), lambda i: (0, 0)),
            pl.BlockSpec((1, 16), lambda i: (0, 0)),
        ],
        out_specs=pl.BlockSpec((32, 16), lambda i: (0, 0)),
        out_shape=jax.ShapeDtypeStruct((32, 16), jnp.float32),
    )(acc2, xs2, c0, c1, bg2r, wc, bcr)


def _deg_kernel(dst_hbm, counts_hbm, acc_sp, zbuf, ones_v, idx_v):
    c = lax.axis_index("c")
    s = lax.axis_index("s")

    def fill_z(i, _):
        zbuf[pl.ds(i * 16, 16)] = jnp.zeros((16,), jnp.float32)
        return 0
    lax.fori_loop(0, 256, fill_z, 0)
    for j in range(8):
        ones_v[pl.ds(j * 16, 16)] = jnp.full((16,), 1.0, jnp.float32)

    pltpu.sync_copy(zbuf, acc_sp.at[pl.ds(s * 4096, 4096)])
    plsc.subcore_barrier()

    def blk(i, _):
        base = c * 4096 + s * 256 + i * 16
        pltpu.sync_copy(dst_hbm.at[pl.ds(base, 16)], idx_v)

        def row(j, _):
            pltpu.sync_copy(ones_v, acc_sp.at[idx_v.at[j]], add=True)
            return 0
        return lax.fori_loop(0, 16, row, 0)
    lax.fori_loop(0, 16, blk, 0)
    plsc.subcore_barrier()
    pltpu.sync_copy(acc_sp.at[pl.ds(s * 4096, 4096)],
                    counts_hbm.at[c, pl.ds(s * 4096, 4096)])


def _deg_call(dst):
    f = functools.partial(
        pl.kernel,
        mesh=plsc.VectorSubcoreMesh(core_axis_name="c", subcore_axis_name="s"),
        compiler_params=pltpu.CompilerParams(use_tc_tiling_on_sc=False),
        out_type=jax.ShapeDtypeStruct((2, N), jnp.float32),
        scratch_types=[
            pltpu.VMEM_SHARED((N,), jnp.float32),
            pltpu.VMEM((4096,), jnp.float32),
            pltpu.VMEM((128,), jnp.float32),
            pltpu.VMEM((16, 128), jnp.int32),
        ],
    )(_deg_kernel)
    return f(dst)


_NBUF = 6
_LOOK = 4


def _scat_kernel(xs4_hbm, src_hbm, dst_hbm, out_hbm,
                 acc_sp, zbuf, sidx, didx, rbuf, gsem, ssem, isem):
    c = lax.axis_index("c")
    s = lax.axis_index("s")
    tbase = c * 4096 + s * 256

    def fill_z(i, _):
        zbuf[i] = jnp.zeros((16,), jnp.float32)
        return 0
    lax.fori_loop(0, 1024, fill_z, 0)

    def issue_idx(i):
        pltpu.async_copy(src_hbm.at[pl.ds(tbase + i * 16, 16)],
                         sidx.at[i % 2], isem)
        pltpu.async_copy(dst_hbm.at[pl.ds(tbase + i * 16, 16)],
                         didx.at[i % 2], isem)

    def wait_idx(i):
        pltpu.make_async_copy(src_hbm.at[pl.ds(tbase + i * 16, 16)],
                              sidx.at[i % 2], isem).wait()
        pltpu.make_async_copy(dst_hbm.at[pl.ds(tbase + i * 16, 16)],
                              didx.at[i % 2], isem).wait()

    for f in range(NCHUNK):
        def zero(i, _):
            pltpu.sync_copy(zbuf, acc_sp.at[pl.ds(s * 4096 + i * 1024, 1024)])
            return 0
        lax.fori_loop(0, 4, zero, 0)
        issue_idx(0)
        plsc.subcore_barrier()

        def blk(i, _):
            ib = i % 2
            wait_idx(i)

            @pl.when(i < 15)
            def _():
                issue_idx(i + 1)

            def gather(t):
                k = t % _NBUF
                return pltpu.async_copy(
                    xs4_hbm.at[f].at[sidx.at[ib, t]], rbuf.at[k], gsem.at[k])

            gh = [None] * 16
            sh = [None] * 16
            for t in range(_LOOK):
                gh[t] = gather(t)
            for j in range(16):
                gh[j].wait()
                sh[j] = pltpu.async_copy(
                    rbuf.at[j % _NBUF], acc_sp.at[didx.at[ib, j]],
                    ssem.at[j % _NBUF], add=True)
                t = j + _LOOK
                if t < 16:
                    if t - _NBUF >= 0:
                        sh[t - _NBUF].wait()
                    gh[t] = gather(t)
            for j in range(16 - _NBUF, 16):
                sh[j].wait()
            return 0
        lax.fori_loop(0, 16, blk, 0)
        plsc.subcore_barrier()
        pltpu.sync_copy(acc_sp.at[pl.ds(s * 4096, 4096)],
                        out_hbm.at[c, f, pl.ds(s * 4096, 4096)])
        plsc.subcore_barrier()


def _scat_call(xs4, src, dst):
    f = functools.partial(
        pl.kernel,
        mesh=plsc.VectorSubcoreMesh(core_axis_name="c", subcore_axis_name="s"),
        compiler_params=pltpu.CompilerParams(use_tc_tiling_on_sc=False),
        out_type=jax.ShapeDtypeStruct((2, NCHUNK, N, FC), jnp.float32),
        scratch_types=[
            pltpu.VMEM_SHARED((N, FC), jnp.float32),
            pltpu.VMEM((1024, FC), jnp.float32),
            pltpu.VMEM((2, 16, 128), jnp.int32),
            pltpu.VMEM((2, 16, 128), jnp.int32),
            pltpu.VMEM((_NBUF, 128, FC), jnp.float32),
            pltpu.SemaphoreType.DMA((_NBUF,)),
            pltpu.SemaphoreType.DMA((_NBUF,)),
            pltpu.SemaphoreType.DMA,
        ],
    )(_scat_kernel)
    return f(xs4, src, dst)


def kernel(x, edge_index, W1, b1, W2, b2, Wg1, bg1, Wg2, bg2, Wc, bc):
    B, C, T = x.shape

    x4 = jnp.transpose(x.reshape(B, C, T // 4, 4), (0, 3, 1, 2))
    w1s = jnp.transpose(W1, (2, 0, 1))
    w2s = jnp.transpose(W2, (2, 0, 1))
    b1c = b1.reshape(16, 1)
    b2c = b2.reshape(32, 1)
    bg1r = bg1.reshape(1, 64)
    bg2r = bg2.reshape(1, 64)
    bcr = bc.reshape(1, 16)
    src = edge_index[0].reshape(ROWS, 128)
    dst = edge_index[1].reshape(ROWS, 128)

    counts = _deg_call(dst)
    c0 = counts[0].reshape(N, 1)
    c1 = counts[1].reshape(N, 1)

    xs1 = _front_call(x4, w1s, b1c, w2s, b2c, c0, c1, Wg1)
    acc1 = _scat_call(xs1, src, dst)
    xs2 = _mid_call(acc1, xs1, c0, c1, bg1r, Wg2)
    acc2 = _scat_call(xs2, src, dst)
    return _final_call(acc2, xs2, c0, c1, bg2r, Wc, bcr)


# speedup vs baseline: 37.0025x; 1.5856x over previous
import functools
import jax
import jax.numpy as jnp
from jax import lax
from jax.experimental import pallas as pl
from jax.experimental.pallas import tpu as pltpu
from jax.experimental.pallas import tpu_sc as plsc

N = 65536
NB = 2048
ROWS = 8192
FC = 32
NCHUNK = 2


def _shift1(x, q):
    if q == 0:
        return x
    zc = jnp.zeros((x.shape[0], 1), jnp.float32)
    if q == -1:
        return jnp.concatenate([zc, x[:, :-1]], axis=1)
    return jnp.concatenate([x[:, 1:], zc], axis=1)


def _front_body(xp_ref, w1_ref, b1_ref, w2_ref, b2_ref, c0_ref, c1_ref,
                wg_ref, out_ref):
    xp = xp_ref[0]
    sh1 = {}
    for r in range(4):
        for q in (-1, 0, 1):
            sh1[(r, q)] = None
    c1 = []
    for p in range(4):
        a = jnp.zeros((16, NB), jnp.float32)
        for k in range(5):
            d = p + k - 2
            r = d % 4
            q = (d - r) // 4
            if sh1[(r, q)] is None:
                sh1[(r, q)] = _shift1(xp[r], q)
            a = a + jnp.dot(w1_ref[k], sh1[(r, q)],
                            preferred_element_type=jnp.float32)
        c1.append(jnp.maximum(a + b1_ref[...], 0.0))
    m = [jnp.maximum(c1[0], c1[1]), jnp.maximum(c1[2], c1[3])]
    sh2 = {}
    c2 = []
    for p in range(2):
        a = jnp.zeros((32, NB), jnp.float32)
        for k in range(5):
            d = p + k - 2
            r = d % 2
            q = (d - r) // 2
            if (r, q) not in sh2:
                sh2[(r, q)] = _shift1(m[r], q)
            a = a + jnp.dot(w2_ref[k], sh2[(r, q)],
                            preferred_element_type=jnp.float32)
        c2.append(jnp.maximum(a + b2_ref[...], 0.0))
    h2 = jnp.maximum(c2[0], c2[1])
    h0b = jnp.transpose(h2, (1, 0))
    dinv = lax.rsqrt(c0_ref[...] + c1_ref[...] + 1.0)
    xs = dinv * jnp.dot(h0b, wg_ref[...], preferred_element_type=jnp.float32)
    for c in range(NCHUNK):
        out_ref[c] = xs[:, c * FC:(c + 1) * FC].astype(jnp.bfloat16)


def _front_call(x4, w1s, b1c, w2s, b2c, c0, c1, wg1):
    return pl.pallas_call(
        _front_body,
        grid=(32,),
        in_specs=[
            pl.BlockSpec((1, 4, 64, NB), lambda b: (b, 0, 0, 0)),
            pl.BlockSpec((5, 16, 64), lambda b: (0, 0, 0)),
            pl.BlockSpec((16, 1), lambda b: (0, 0)),
            pl.BlockSpec((5, 32, 16), lambda b: (0, 0, 0)),
            pl.BlockSpec((32, 1), lambda b: (0, 0)),
            pl.BlockSpec((NB, 1), lambda b: (b, 0)),
            pl.BlockSpec((NB, 1), lambda b: (b, 0)),
            pl.BlockSpec((32, 64), lambda b: (0, 0)),
        ],
        out_specs=pl.BlockSpec((NCHUNK, NB, FC), lambda b: (0, b, 0)),
        out_shape=jax.ShapeDtypeStruct((NCHUNK, N, FC), jnp.bfloat16),
    )(x4, w1s, b1c, w2s, b2c, c0, c1, wg1)


def _mid_body(acc_ref, xs_ref, c0_ref, c1_ref, bg_ref, wg_ref, out_ref):
    dinv = lax.rsqrt(c0_ref[...] + c1_ref[...] + 1.0)
    s = jnp.concatenate(
        [acc_ref[0, i].astype(jnp.float32) + acc_ref[1, i].astype(jnp.float32)
         + xs_ref[i].astype(jnp.float32) for i in range(NCHUNK)],
        axis=1)
    h = jnp.maximum(dinv * s + bg_ref[...], 0.0)
    y = jnp.dot(h, wg_ref[...], preferred_element_type=jnp.float32)
    ys = dinv * y
    for c in range(NCHUNK):
        out_ref[c] = ys[:, c * FC:(c + 1) * FC].astype(jnp.bfloat16)


def _mid_call(acc1, xs1, c0, c1, bg1r, wg2):
    return pl.pallas_call(
        _mid_body,
        grid=(N // NB,),
        in_specs=[
            pl.BlockSpec((2, NCHUNK, NB, FC), lambda i: (0, 0, i, 0)),
            pl.BlockSpec((NCHUNK, NB, FC), lambda i: (0, i, 0)),
            pl.BlockSpec((NB, 1), lambda i: (i, 0)),
            pl.BlockSpec((NB, 1), lambda i: (i, 0)),
            pl.BlockSpec((1, 64), lambda i: (0, 0)),
            pl.BlockSpec((64, 64), lambda i: (0, 0)),
        ],
        out_specs=pl.BlockSpec((NCHUNK, NB, FC), lambda i: (0, i, 0)),
        out_shape=jax.ShapeDtypeStruct((NCHUNK, N, FC), jnp.bfloat16),
    )(acc1, xs1, c0, c1, bg1r, wg2)


def _final_body(acc_ref, xs_ref, c0_ref, c1_ref, bg_ref, wc_ref, bc_ref,
                out_ref):
    b = pl.program_id(0)
    dinv = lax.rsqrt(c0_ref[...] + c1_ref[...] + 1.0)
    s = jnp.concatenate(
        [acc_ref[0, i].astype(jnp.float32) + acc_ref[1, i].astype(jnp.float32)
         + xs_ref[i].astype(jnp.float32) for i in range(NCHUNK)],
        axis=1)
    h = jnp.maximum(dinv * s + bg_ref[...], 0.0)
    hm = jnp.mean(h, axis=0, keepdims=True)
    out_ref[pl.ds(b, 1), :] = (
        jnp.dot(hm, wc_ref[...], preferred_element_type=jnp.float32)
        + bc_ref[...])


def _final_call(acc2, xs2, c0, c1, bg2r, wc, bcr):
    return pl.pallas_call(
        _final_body,
        grid=(32,),
        in_specs=[
            pl.BlockSpec((2, NCHUNK, NB, FC), lambda i: (0, 0, i, 0)),
            pl.BlockSpec((NCHUNK, NB, FC), lambda i: (0, i, 0)),
            pl.BlockSpec((NB, 1), lambda i: (i, 0)),
            pl.BlockSpec((NB, 1), lambda i: (i, 0)),
            pl.BlockSpec((1, 64), lambda i: (0, 0)),
            pl.BlockSpec((64, 16), lambda i: (0, 0)),
            pl.BlockSpec((1, 16), lambda i: (0, 0)),
        ],
        out_specs=pl.BlockSpec((32, 16), lambda i: (0, 0)),
        out_shape=jax.ShapeDtypeStruct((32, 16), jnp.float32),
    )(acc2, xs2, c0, c1, bg2r, wc, bcr)


def _deg_kernel(dst_hbm, counts_hbm, acc_sp, zbuf, ones_v, idx_v):
    c = lax.axis_index("c")
    s = lax.axis_index("s")

    def fill_z(i, _):
        zbuf[pl.ds(i * 16, 16)] = jnp.zeros((16,), jnp.float32)
        return 0
    lax.fori_loop(0, 256, fill_z, 0)
    for j in range(8):
        ones_v[pl.ds(j * 16, 16)] = jnp.full((16,), 1.0, jnp.float32)

    pltpu.sync_copy(zbuf, acc_sp.at[pl.ds(s * 4096, 4096)])
    plsc.subcore_barrier()

    def blk(i, _):
        base = c * 4096 + s * 256 + i * 16
        pltpu.sync_copy(dst_hbm.at[pl.ds(base, 16)], idx_v)

        def row(j, _):
            pltpu.sync_copy(ones_v, acc_sp.at[idx_v.at[j]], add=True)
            return 0
        return lax.fori_loop(0, 16, row, 0)
    lax.fori_loop(0, 16, blk, 0)
    plsc.subcore_barrier()
    pltpu.sync_copy(acc_sp.at[pl.ds(s * 4096, 4096)],
                    counts_hbm.at[c, pl.ds(s * 4096, 4096)])


def _deg_call(dst):
    f = functools.partial(
        pl.kernel,
        mesh=plsc.VectorSubcoreMesh(core_axis_name="c", subcore_axis_name="s"),
        compiler_params=pltpu.CompilerParams(use_tc_tiling_on_sc=False),
        out_type=jax.ShapeDtypeStruct((2, N), jnp.float32),
        scratch_types=[
            pltpu.VMEM_SHARED((N,), jnp.float32),
            pltpu.VMEM((4096,), jnp.float32),
            pltpu.VMEM((128,), jnp.float32),
            pltpu.VMEM((16, 128), jnp.int32),
        ],
    )(_deg_kernel)
    return f(dst)


_NBUF = 6
_LOOK = 4


def _scat_kernel(xs4_hbm, src_hbm, dst_hbm, out_hbm,
                 acc_sp, zbuf, sidx, didx, rbuf, gsem, ssem, isem):
    c = lax.axis_index("c")
    s = lax.axis_index("s")
    tbase = c * 4096 + s * 256

    def fill_z(i, _):
        zbuf[i] = jnp.zeros((32,), jnp.bfloat16)
        return 0
    lax.fori_loop(0, 1024, fill_z, 0)

    def issue_idx(i):
        pltpu.async_copy(src_hbm.at[pl.ds(tbase + i * 16, 16)],
                         sidx.at[i % 2], isem)
        pltpu.async_copy(dst_hbm.at[pl.ds(tbase + i * 16, 16)],
                         didx.at[i % 2], isem)

    def wait_idx(i):
        pltpu.make_async_copy(src_hbm.at[pl.ds(tbase + i * 16, 16)],
                              sidx.at[i % 2], isem).wait()
        pltpu.make_async_copy(dst_hbm.at[pl.ds(tbase + i * 16, 16)],
                              didx.at[i % 2], isem).wait()

    for f in range(NCHUNK):
        def zero(i, _):
            pltpu.sync_copy(zbuf, acc_sp.at[pl.ds(s * 4096 + i * 1024, 1024)])
            return 0
        lax.fori_loop(0, 4, zero, 0)
        issue_idx(0)
        plsc.subcore_barrier()

        def blk(i, _):
            ib = i % 2
            wait_idx(i)

            @pl.when(i < 15)
            def _():
                issue_idx(i + 1)

            def gather(t):
                k = t % _NBUF
                return pltpu.async_copy(
                    xs4_hbm.at[f].at[sidx.at[ib, t]], rbuf.at[k], gsem.at[k])

            gh = [None] * 16
            sh = [None] * 16
            for t in range(_LOOK):
                gh[t] = gather(t)
            for j in range(16):
                gh[j].wait()
                sh[j] = pltpu.async_copy(
                    rbuf.at[j % _NBUF], acc_sp.at[didx.at[ib, j]],
                    ssem.at[j % _NBUF], add=True)
                t = j + _LOOK
                if t < 16:
                    if t - _NBUF >= 0:
                        sh[t - _NBUF].wait()
                    gh[t] = gather(t)
            for j in range(16 - _NBUF, 16):
                sh[j].wait()
            return 0
        lax.fori_loop(0, 16, blk, 0)
        plsc.subcore_barrier()
        pltpu.sync_copy(acc_sp.at[pl.ds(s * 4096, 4096)],
                        out_hbm.at[c, f, pl.ds(s * 4096, 4096)])
        plsc.subcore_barrier()


def _scat_call(xs4, src, dst):
    f = functools.partial(
        pl.kernel,
        mesh=plsc.VectorSubcoreMesh(core_axis_name="c", subcore_axis_name="s"),
        compiler_params=pltpu.CompilerParams(use_tc_tiling_on_sc=False),
        out_type=jax.ShapeDtypeStruct((2, NCHUNK, N, FC), jnp.bfloat16),
        scratch_types=[
            pltpu.VMEM_SHARED((N, FC), jnp.bfloat16),
            pltpu.VMEM((1024, FC), jnp.bfloat16),
            pltpu.VMEM((2, 16, 128), jnp.int32),
            pltpu.VMEM((2, 16, 128), jnp.int32),
            pltpu.VMEM((_NBUF, 128, FC), jnp.bfloat16),
            pltpu.SemaphoreType.DMA((_NBUF,)),
            pltpu.SemaphoreType.DMA((_NBUF,)),
            pltpu.SemaphoreType.DMA,
        ],
    )(_scat_kernel)
    return f(xs4, src, dst)


def kernel(x, edge_index, W1, b1, W2, b2, Wg1, bg1, Wg2, bg2, Wc, bc):
    B, C, T = x.shape

    x4 = jnp.transpose(x.reshape(B, C, T // 4, 4), (0, 3, 1, 2))
    w1s = jnp.transpose(W1, (2, 0, 1))
    w2s = jnp.transpose(W2, (2, 0, 1))
    b1c = b1.reshape(16, 1)
    b2c = b2.reshape(32, 1)
    bg1r = bg1.reshape(1, 64)
    bg2r = bg2.reshape(1, 64)
    bcr = bc.reshape(1, 16)
    src = edge_index[0].reshape(ROWS, 128)
    dst = edge_index[1].reshape(ROWS, 128)

    counts = _deg_call(dst)
    c0 = counts[0].reshape(N, 1)
    c1 = counts[1].reshape(N, 1)

    xs1 = _front_call(x4, w1s, b1c, w2s, b2c, c0, c1, Wg1)
    acc1 = _scat_call(xs1, src, dst)
    xs2 = _mid_call(acc1, xs1, c0, c1, bg1r, Wg2)
    acc2 = _scat_call(xs2, src, dst)
    return _final_call(acc2, xs2, c0, c1, bg2r, Wc, bcr)
